# baseline (speedup 1.0000x reference)
import functools
import math

import jax
import jax.numpy as jnp
from jax import lax
from jax.experimental import pallas as pl
from jax.experimental.pallas import tpu as pltpu
from jax.experimental.pallas import tpu_sc as plsc

EMB = 64
SCALE = math.sqrt(EMB)
LANES = 16

_NC = 2
_NS = 16
_NW = _NC * _NS

_CHUNK = 512


def _make_gather(B: int):
    per_w = B // _NW
    n_chunks = per_w // _CHUNK
    mesh = plsc.VectorSubcoreMesh(core_axis_name="c", subcore_axis_name="s")

    @functools.partial(
        pl.kernel,
        mesh=mesh,
        compiler_params=pltpu.CompilerParams(use_tc_tiling_on_sc=False),
        out_type=jax.ShapeDtypeStruct((B, EMB), jnp.float32),
        scratch_types=[
            pltpu.VMEM((_CHUNK,), jnp.int32),
            pltpu.VMEM((_CHUNK, EMB), jnp.float32),
            pltpu.SemaphoreType.DMA,
        ],
    )
    def gather_kernel(idx_hbm, table_hbm, out_hbm, idx_v, rows_v, sem):
        wid = lax.axis_index("s") * _NC + lax.axis_index("c")
        base = wid * per_w

        def chunk_body(g, carry):
            off = base + g * _CHUNK
            pltpu.sync_copy(idx_hbm.at[pl.ds(off, _CHUNK)], idx_v)
            pltpu.async_copy(table_hbm.at[idx_v], rows_v, sem).wait()

            def scale_body(i, c):
                for j in range(EMB // LANES):
                    sl = pl.ds(j * LANES, LANES)
                    rows_v[i, sl] = rows_v[i, sl] * SCALE
                return c

            lax.fori_loop(0, _CHUNK, scale_body, 0)
            pltpu.sync_copy(rows_v, out_hbm.at[pl.ds(off, _CHUNK)])
            return carry

        lax.fori_loop(0, n_chunks, chunk_body, 0)

    return gather_kernel


def kernel(tokens, embedding_weight):
    b, t = tokens.shape
    flat_idx = tokens.reshape(-1).astype(jnp.int32)
    out = _make_gather(b * t)(flat_idx, embedding_weight)
    return out.reshape(b, t, EMB)


# speedup vs baseline: 1.1301x; 1.1301x over previous
import functools
import math

import jax
import jax.numpy as jnp
from jax import lax
from jax.experimental import pallas as pl
from jax.experimental.pallas import tpu as pltpu
from jax.experimental.pallas import tpu_sc as plsc

EMB = 64
SCALE = math.sqrt(EMB)
LANES = 16

_NC = 2
_NS = 16
_NW = _NC * _NS

_CHUNK = 800
_NBUF = 2
_ROWS_PER_IT = 4


def _make_gather(B: int):
    per_w = B // _NW
    n_chunks = per_w // _CHUNK
    n_main = n_chunks // _NBUF - 1
    mesh = plsc.VectorSubcoreMesh(core_axis_name="c", subcore_axis_name="s")

    @functools.partial(
        pl.kernel,
        mesh=mesh,
        compiler_params=pltpu.CompilerParams(use_tc_tiling_on_sc=False),
        out_type=jax.ShapeDtypeStruct((B, EMB), jnp.float32),
        scratch_types=[
            pltpu.VMEM((_NBUF, _CHUNK), jnp.int32),
            pltpu.VMEM((_NBUF, _CHUNK, EMB), jnp.float32),
            [pltpu.SemaphoreType.DMA] * _NBUF,
            [pltpu.SemaphoreType.DMA] * _NBUF,
        ],
    )
    def gather_kernel(idx_hbm, table_hbm, out_hbm, idx_v, rows_v, g_sems, o_sems):
        wid = lax.axis_index("s") * _NC + lax.axis_index("c")
        base = wid * per_w

        def start_gather(b, off):
            pltpu.sync_copy(idx_hbm.at[pl.ds(off, _CHUNK)], idx_v.at[b])
            pltpu.async_copy(table_hbm.at[idx_v.at[b]], rows_v.at[b], g_sems[b])

        def wait_gather(b):
            pltpu.make_async_copy(
                table_hbm.at[idx_v.at[b]], rows_v.at[b], g_sems[b]
            ).wait()

        def scale_chunk(b):
            rows_b = rows_v.at[b]

            def scale_body(i, c):
                for r in range(_ROWS_PER_IT):
                    for j in range(EMB // LANES):
                        sl = pl.ds(j * LANES, LANES)
                        row = i * _ROWS_PER_IT + r
                        rows_b[row, sl] = rows_b[row, sl] * SCALE
                return c

            lax.fori_loop(0, _CHUNK // _ROWS_PER_IT, scale_body, 0)

        def start_out(b, off):
            pltpu.async_copy(
                rows_v.at[b], out_hbm.at[pl.ds(off, _CHUNK)], o_sems[b]
            )

        def wait_out(b, off):
            pltpu.make_async_copy(
                rows_v.at[b], out_hbm.at[pl.ds(off, _CHUNK)], o_sems[b]
            ).wait()

        for b in range(_NBUF):
            start_gather(b, base + b * _CHUNK)

        def main_body(it, carry):
            off0 = base + it * _NBUF * _CHUNK
            for b in range(_NBUF):
                wait_gather(b)
                scale_chunk(b)
                start_out(b, off0 + b * _CHUNK)
            for b in range(_NBUF):
                wait_out(b, off0 + b * _CHUNK)
                start_gather(b, off0 + (_NBUF + b) * _CHUNK)
            return carry

        lax.fori_loop(0, n_main, main_body, 0)

        off0 = base + n_main * _NBUF * _CHUNK
        for b in range(_NBUF):
            wait_gather(b)
            scale_chunk(b)
            start_out(b, off0 + b * _CHUNK)
        for b in range(_NBUF):
            wait_out(b, off0 + b * _CHUNK)

    return gather_kernel


def kernel(tokens, embedding_weight):
    b, t = tokens.shape
    flat_idx = tokens.reshape(-1).astype(jnp.int32)
    out = _make_gather(b * t)(flat_idx, embedding_weight)
    return out.reshape(b, t, EMB)


# speedup vs baseline: 1.1324x; 1.0020x over previous
import functools
import math

import jax
import jax.numpy as jnp
from jax import lax
from jax.experimental import pallas as pl
from jax.experimental.pallas import tpu as pltpu
from jax.experimental.pallas import tpu_sc as plsc

EMB = 64
SCALE = math.sqrt(EMB)
LANES = 16

_NC = 2
_NS = 16
_NW = _NC * _NS

_CHUNK = 800
_NBUF = 2
_ROWS_PER_IT = 4


def _make_gather(B: int):
    per_w = B // _NW
    n_chunks = per_w // _CHUNK
    n_main = n_chunks // _NBUF - 1
    mesh = plsc.VectorSubcoreMesh(core_axis_name="c", subcore_axis_name="s")

    @functools.partial(
        pl.kernel,
        mesh=mesh,
        compiler_params=pltpu.CompilerParams(
            use_tc_tiling_on_sc=False,
            skip_device_barrier=True,
            disable_bounds_checks=True,
            disable_semaphore_checks=True,
        ),
        out_type=jax.ShapeDtypeStruct((B, EMB), jnp.float32),
        scratch_types=[
            pltpu.VMEM((_NBUF, _CHUNK), jnp.int32),
            pltpu.VMEM((_NBUF, _CHUNK, EMB), jnp.float32),
            [pltpu.SemaphoreType.DMA] * _NBUF,
            [pltpu.SemaphoreType.DMA] * _NBUF,
        ],
    )
    def gather_kernel(idx_hbm, table_hbm, out_hbm, idx_v, rows_v, g_sems, o_sems):
        wid = lax.axis_index("s") * _NC + lax.axis_index("c")
        base = wid * per_w

        def start_gather(b, off):
            pltpu.sync_copy(idx_hbm.at[pl.ds(off, _CHUNK)], idx_v.at[b])
            pltpu.async_copy(table_hbm.at[idx_v.at[b]], rows_v.at[b], g_sems[b])

        def wait_gather(b):
            pltpu.make_async_copy(
                table_hbm.at[idx_v.at[b]], rows_v.at[b], g_sems[b]
            ).wait()

        def scale_chunk(b):
            rows_b = rows_v.at[b]

            def scale_body(i, c):
                for r in range(_ROWS_PER_IT):
                    for j in range(EMB // LANES):
                        sl = pl.ds(j * LANES, LANES)
                        row = i * _ROWS_PER_IT + r
                        rows_b[row, sl] = rows_b[row, sl] * SCALE
                return c

            lax.fori_loop(0, _CHUNK // _ROWS_PER_IT, scale_body, 0)

        def start_out(b, off):
            pltpu.async_copy(
                rows_v.at[b], out_hbm.at[pl.ds(off, _CHUNK)], o_sems[b]
            )

        def wait_out(b, off):
            pltpu.make_async_copy(
                rows_v.at[b], out_hbm.at[pl.ds(off, _CHUNK)], o_sems[b]
            ).wait()

        for b in range(_NBUF):
            start_gather(b, base + b * _CHUNK)

        def main_body(it, carry):
            off0 = base + it * _NBUF * _CHUNK
            for b in range(_NBUF):
                wait_gather(b)
                scale_chunk(b)
                start_out(b, off0 + b * _CHUNK)
            for b in range(_NBUF):
                wait_out(b, off0 + b * _CHUNK)
                start_gather(b, off0 + (_NBUF + b) * _CHUNK)
            return carry

        lax.fori_loop(0, n_main, main_body, 0)

        off0 = base + n_main * _NBUF * _CHUNK
        for b in range(_NBUF):
            wait_gather(b)
            scale_chunk(b)
            start_out(b, off0 + b * _CHUNK)
        for b in range(_NBUF):
            wait_out(b, off0 + b * _CHUNK)

    return gather_kernel


def kernel(tokens, embedding_weight):
    b, t = tokens.shape
    flat_idx = tokens.reshape(-1).astype(jnp.int32)
    out = _make_gather(b * t)(flat_idx, embedding_weight)
    return out.reshape(b, t, EMB)
